# baseline (speedup 1.0000x reference)
import functools

import jax
import jax.numpy as jnp
from jax import lax
from jax.experimental import pallas as pl
from jax.experimental.pallas import tpu as pltpu
from jax.experimental.pallas import tpu_sc as plsc

_L = 16
_NBUF = 4
_G = 100
_W = 128


@functools.cache
def _build(B, S, V, D, n_workers):
    bat_w = B // n_workers
    pack = _W // D
    orow_c = S // pack
    out_rows = B * orow_c
    mesh = plsc.VectorSubcoreMesh(core_axis_name="c", subcore_axis_name="s")

    @functools.partial(
        pl.kernel,
        mesh=mesh,
        compiler_params=pltpu.CompilerParams(
            use_tc_tiling_on_sc=False, needs_layout_passes=False),
        out_type=jax.ShapeDtypeStruct((out_rows, _W), jnp.float32),
        scratch_types=[
            pltpu.VMEM((bat_w, S), jnp.int32),
            pltpu.VMEM((2 * bat_w, _G), jnp.int32),
            [pltpu.VMEM((S, D), jnp.float32) for _ in range(_NBUF)],
            [pltpu.VMEM((orow_c, _W), jnp.float32) for _ in range(_NBUF)],
            pltpu.VMEM((S, D), jnp.float32),
            [pltpu.SemaphoreType.DMA for _ in range(_NBUF)],
            [pltpu.SemaphoreType.DMA for _ in range(_NBUF)],
        ],
    )
    def emb(tok_hbm, tokg_hbm, tab_hbm, pos_hbm, out_hbm,
            toka, tokg, rows, obuf, posv, gsem, osem):
        wid = lax.axis_index("s") * 2 + lax.axis_index("c")
        bbase = wid * bat_w
        n_grp = S // _L
        tail = S - n_grp * _L

        pltpu.sync_copy(pos_hbm, posv)
        pltpu.sync_copy(tok_hbm.at[wid], toka)
        pltpu.sync_copy(tokg_hbm.at[wid], tokg)

        def gather(bi, b):
            pltpu.make_async_copy(
                tab_hbm.at[tokg.at[2 * bi]],
                rows[b].at[pl.ds(0, _G)], gsem[b]).start()
            pltpu.make_async_copy(
                tab_hbm.at[tokg.at[2 * bi + 1]],
                rows[b].at[pl.ds(_G, _G)], gsem[b]).start()

        def gwait(bi, b):
            pltpu.make_async_copy(
                tab_hbm.at[tokg.at[2 * bi]],
                rows[b].at[pl.ds(0, _G)], gsem[b]).wait()
            pltpu.make_async_copy(
                tab_hbm.at[tokg.at[2 * bi + 1]],
                rows[b].at[pl.ds(_G, _G)], gsem[b]).wait()

        def compute(bi, b):
            def do_row(i, tvec, k):
                t = tvec[k]
                for d in range(D // _L):
                    p = posv[i, pl.ds(d * _L, _L)]
                    r = rows[b][i, pl.ds(d * _L, _L)]
                    osl = pl.ds((i % pack) * D + d * _L, _L)
                    obuf[b][i // pack, osl] = jnp.where(t == 0, p, r + p)

            for g in range(n_grp):
                tvec = toka[bi, pl.ds(g * _L, _L)]
                for k in range(_L):
                    do_row(g * _L + k, tvec, k)
            if tail:
                off = S - _L
                tvec = toka[bi, pl.ds(off, _L)]
                for k in range(_L - tail, _L):
                    do_row(off + k, tvec, k)

        gather(0, 0)
        gather(1, 1)

        def quad_body(q, _):
            for b in range(_NBUF):
                bi = q * _NBUF + b
                gwait(bi, b)
                compute(bi, b)
                pltpu.make_async_copy(
                    obuf[b],
                    out_hbm.at[pl.ds((bbase + bi) * orow_c, orow_c)],
                    osem[b]).start()
                bn = (b + 2) % _NBUF

                @pl.when(bi >= 2)
                def _():
                    pltpu.make_async_copy(
                        obuf[bn],
                        out_hbm.at[pl.ds((bbase + bi - 2) * orow_c, orow_c)],
                        osem[bn]).wait()

                @pl.when(bi + 2 < bat_w)
                def _():
                    gather(bi + 2, bn)
            return 0

        lax.fori_loop(0, bat_w // _NBUF, quad_body, 0)
        for bi in (bat_w - 2, bat_w - 1):
            b = bi % _NBUF
            pltpu.make_async_copy(
                obuf[b],
                out_hbm.at[pl.ds((bbase + bi) * orow_c, orow_c)],
                osem[b]).wait()

    return emb


def kernel(melody_tokens, chord_tokens, melody_embedding, chord_embedding,
           encoder_position, decoder_position):
    B, S = melody_tokens.shape
    V, D = melody_embedding.shape
    n_workers = 32
    bat_w = B // n_workers
    emb = _build(B, S, V, D, n_workers)
    mel = melody_tokens.astype(jnp.int32)
    cho = chord_tokens.astype(jnp.int32)
    mo = emb(mel.reshape(n_workers, bat_w, S),
             mel.reshape(n_workers, 2 * bat_w, _G),
             melody_embedding, encoder_position[:S])
    co = emb(cho.reshape(n_workers, bat_w, S),
             cho.reshape(n_workers, 2 * bat_w, _G),
             chord_embedding, decoder_position[:S])
    return mo.reshape(B, S, D), co.reshape(B, S, D)


# speedup vs baseline: 1.0330x; 1.0330x over previous
import functools

import jax
import jax.numpy as jnp
from jax import lax
from jax.experimental import pallas as pl
from jax.experimental.pallas import tpu as pltpu
from jax.experimental.pallas import tpu_sc as plsc

_L = 16
_NBUF = 4


@functools.cache
def _build(B, S, V, D, n_workers):
    total = B * S
    bat_w = B // n_workers
    per_w = total // n_workers
    mesh = plsc.VectorSubcoreMesh(core_axis_name="c", subcore_axis_name="s")

    @functools.partial(
        pl.kernel,
        mesh=mesh,
        compiler_params=pltpu.CompilerParams(
            use_tc_tiling_on_sc=False, needs_layout_passes=False),
        out_type=jax.ShapeDtypeStruct((total, D), jnp.float32),
        scratch_types=[
            pltpu.VMEM((bat_w, S), jnp.int32),
            pltpu.VMEM((S, bat_w), jnp.int32),
            pltpu.VMEM((S, bat_w), jnp.int32),
            [pltpu.VMEM((bat_w, D), jnp.float32) for _ in range(_NBUF)],
            [pltpu.VMEM((bat_w, D), jnp.float32) for _ in range(_NBUF)],
            pltpu.VMEM((S, D), jnp.float32),
            [pltpu.SemaphoreType.DMA for _ in range(_NBUF)],
            [pltpu.SemaphoreType.DMA for _ in range(_NBUF)],
        ],
    )
    def emb(tok_hbm, tab_hbm, pos_hbm, out_hbm,
            toka, tokt, idxt, rows, obuf, posv, gsem, osem):
        wid = lax.axis_index("s") * 2 + lax.axis_index("c")
        base = wid * per_w
        lanes = lax.iota(jnp.int32, _L)

        pltpu.sync_copy(pos_hbm, posv)
        pltpu.sync_copy(tok_hbm.at[wid], toka)

        def tr_body(s, _):
            for g in range(bat_w // _L):
                col = plsc.load_gather(
                    toka, [g * _L + lanes, jnp.full((_L,), s, jnp.int32)])
                tokt[s, pl.ds(g * _L, _L)] = col
                idxt[s, pl.ds(g * _L, _L)] = base + (g * _L + lanes) * S + s
            return 0
        lax.fori_loop(0, S, tr_body, 0)

        def gather(s, b):
            pltpu.make_async_copy(
                tab_hbm.at[tokt.at[s]], rows[b], gsem[b]).start()

        def compute(s, b):
            p = [posv[s, pl.ds(d * _L, _L)] for d in range(D // _L)]
            for g in range(bat_w // _L):
                tvec = tokt[s, pl.ds(g * _L, _L)]
                for k in range(_L):
                    t = tvec[k]
                    i = g * _L + k
                    for d in range(D // _L):
                        sl = pl.ds(d * _L, _L)
                        obuf[b][i, sl] = jnp.where(
                            t == 0, p[d], rows[b][i, sl] + p[d])

        gather(0, 0)
        gather(1, 1)

        def quad_body(q, _):
            for b in range(_NBUF):
                s = q * _NBUF + b
                pltpu.make_async_copy(
                    tab_hbm.at[tokt.at[s]], rows[b], gsem[b]).wait()
                compute(s, b)
                pltpu.make_async_copy(
                    obuf[b], out_hbm.at[idxt.at[s]], osem[b]).start()
                bn = (b + 2) % _NBUF

                @pl.when(s >= 2)
                def _():
                    pltpu.make_async_copy(
                        obuf[bn], out_hbm.at[idxt.at[s - 2]],
                        osem[bn]).wait()

                @pl.when(s + 2 < S)
                def _():
                    gather(s + 2, bn)
            return 0

        lax.fori_loop(0, S // _NBUF, quad_body, 0)
        for s in (S - 2, S - 1):
            b = s % _NBUF
            pltpu.make_async_copy(
                obuf[b], out_hbm.at[idxt.at[s]], osem[b]).wait()

    return emb


def kernel(melody_tokens, chord_tokens, melody_embedding, chord_embedding,
           encoder_position, decoder_position):
    B, S = melody_tokens.shape
    V, D = melody_embedding.shape
    n_workers = 32
    emb = _build(B, S, V, D, n_workers)
    mel = melody_tokens.astype(jnp.int32).reshape(n_workers, B // n_workers, S)
    cho = chord_tokens.astype(jnp.int32).reshape(n_workers, B // n_workers, S)
    mo = emb(mel, melody_embedding, encoder_position[:S])
    co = emb(cho, chord_embedding, decoder_position[:S])
    return mo.reshape(B, S, D), co.reshape(B, S, D)
